# Initial kernel scaffold; baseline (speedup 1.0000x reference)
import jax
import jax.numpy as jnp
from jax.experimental import pallas as pl


def kernel(data, emb_table):
    raise NotImplementedError("write your pallas kernel here")


# speedup vs baseline: 6.4385x; 6.4385x over previous
import jax
import jax.numpy as jnp
from jax.experimental import pallas as pl
from jax.experimental.pallas import tpu as pltpu

FEAT = 115
EMB = 13
T = 13
OUT = FEAT + EMB

BLOCK_ROWS = 500


def _concat_kernel(data_ref, embp_ref, out_ref):
    d = data_ref[...]
    dpad = jnp.pad(d, ((0, 0), (0, 0), (0, EMB)))
    out_ref[...] = dpad + embp_ref[...]


def kernel(data, emb_table):
    g, n, t, f = data.shape
    rows = g * n
    data2 = data.reshape(rows, t, f)
    embp = jnp.pad(emb_table, ((0, 0), (FEAT, 0)))[None]

    out = pl.pallas_call(
        _concat_kernel,
        grid=(rows // BLOCK_ROWS,),
        in_specs=[
            pl.BlockSpec((BLOCK_ROWS, t, f), lambda i: (i, 0, 0)),
            pl.BlockSpec((1, t, OUT), lambda i: (0, 0, 0)),
        ],
        out_specs=pl.BlockSpec((BLOCK_ROWS, t, OUT), lambda i: (i, 0, 0)),
        out_shape=jax.ShapeDtypeStruct((rows, t, OUT), data.dtype),
    )(data2, embp)
    return out.reshape(g, n, t, OUT)


# speedup vs baseline: 6.4487x; 1.0016x over previous
import jax
import jax.numpy as jnp
from jax.experimental import pallas as pl
from jax.experimental.pallas import tpu as pltpu

FEAT = 115
EMB = 13
T = 13
OUT = FEAT + EMB

BLOCK_ROWS = 1000


def _concat_kernel(data_ref, embp_ref, out_ref):
    d = data_ref[...]
    dpad = jnp.pad(d, ((0, 0), (0, 0), (0, EMB)))
    out_ref[...] = dpad + embp_ref[...]


def kernel(data, emb_table):
    g, n, t, f = data.shape
    rows = g * n
    data2 = data.reshape(rows, t, f)
    embp = jnp.pad(emb_table, ((0, 0), (FEAT, 0)))[None]

    out = pl.pallas_call(
        _concat_kernel,
        grid=(rows // BLOCK_ROWS,),
        in_specs=[
            pl.BlockSpec((BLOCK_ROWS, t, f), lambda i: (i, 0, 0)),
            pl.BlockSpec((1, t, OUT), lambda i: (0, 0, 0)),
        ],
        out_specs=pl.BlockSpec((BLOCK_ROWS, t, OUT), lambda i: (i, 0, 0)),
        out_shape=jax.ShapeDtypeStruct((rows, t, OUT), data.dtype),
    )(data2, embp)
    return out.reshape(g, n, t, OUT)


# speedup vs baseline: 7.7975x; 1.2092x over previous
import jax
import jax.numpy as jnp
from jax.experimental import pallas as pl

FEAT = 115
EMB = 13
T = 13
OUT = FEAT + EMB
IN_ROW = T * FEAT
OUT_ROW = T * OUT

BLOCK_ROWS = 1000


def _concat_kernel(data_ref, embp_ref, out_ref):
    d = data_ref[...]
    parts = [
        jnp.pad(d[:, t * FEAT:(t + 1) * FEAT], ((0, 0), (0, EMB)))
        for t in range(T)
    ]
    out_ref[...] = jnp.concatenate(parts, axis=1) + embp_ref[...]


def kernel(data, emb_table):
    g, n, t, f = data.shape
    rows = g * n
    data2 = data.reshape(rows, IN_ROW)
    embp = jnp.pad(emb_table, ((0, 0), (FEAT, 0))).reshape(1, OUT_ROW)

    out = pl.pallas_call(
        _concat_kernel,
        grid=(rows // BLOCK_ROWS,),
        in_specs=[
            pl.BlockSpec((BLOCK_ROWS, IN_ROW), lambda i: (i, 0)),
            pl.BlockSpec((1, OUT_ROW), lambda i: (0, 0)),
        ],
        out_specs=pl.BlockSpec((BLOCK_ROWS, OUT_ROW), lambda i: (i, 0)),
        out_shape=jax.ShapeDtypeStruct((rows, OUT_ROW), data.dtype),
    )(data2, embp)
    return out.reshape(g, n, t, OUT)


# speedup vs baseline: 15.3770x; 1.9720x over previous
import jax
import jax.numpy as jnp
from jax.experimental import pallas as pl

FEAT = 115
EMB = 13
T = 13
OUT = FEAT + EMB

BLOCK_N = 1000


def _concat_kernel(data_ref, embp_ref, out_ref):
    d = data_ref[...]
    dpad = jnp.pad(d, ((0, 0), (0, 0), (0, 0), (0, EMB)))
    out_ref[...] = dpad + embp_ref[...]


def kernel(data, emb_table):
    g, n, t, f = data.shape
    embp = jnp.pad(emb_table, ((0, 0), (FEAT, 0)))[None, None]

    return pl.pallas_call(
        _concat_kernel,
        grid=(g, n // BLOCK_N),
        in_specs=[
            pl.BlockSpec((1, BLOCK_N, t, f), lambda gi, i: (gi, i, 0, 0)),
            pl.BlockSpec((1, 1, t, OUT), lambda gi, i: (0, 0, 0, 0)),
        ],
        out_specs=pl.BlockSpec((1, BLOCK_N, t, OUT), lambda gi, i: (gi, i, 0, 0)),
        out_shape=jax.ShapeDtypeStruct((g, n, t, OUT), data.dtype),
    )(data, embp)


# speedup vs baseline: 15.4295x; 1.0034x over previous
import jax
import jax.numpy as jnp
from jax.experimental import pallas as pl
from jax.experimental.pallas import tpu as pltpu

FEAT = 115
EMB = 13
T = 13
OUT = FEAT + EMB

BLOCK_N = 1000


def _concat_kernel(data_ref, embp_ref, out_ref):
    d = data_ref[...]
    dpad = jnp.pad(d, ((0, 0), (0, 0), (0, 0), (0, EMB)))
    out_ref[...] = dpad + embp_ref[...]


def kernel(data, emb_table):
    g, n, t, f = data.shape
    embp = jnp.pad(emb_table, ((0, 0), (FEAT, 0)))[None, None]

    return pl.pallas_call(
        _concat_kernel,
        grid=(g, n // BLOCK_N),
        in_specs=[
            pl.BlockSpec((1, BLOCK_N, t, f), lambda gi, i: (gi, i, 0, 0)),
            pl.BlockSpec((1, 1, t, OUT), lambda gi, i: (0, 0, 0, 0)),
        ],
        out_specs=pl.BlockSpec((1, BLOCK_N, t, OUT), lambda gi, i: (gi, i, 0, 0)),
        out_shape=jax.ShapeDtypeStruct((g, n, t, OUT), data.dtype),
        compiler_params=pltpu.CompilerParams(
            dimension_semantics=("parallel", "parallel")),
    )(data, embp)
